# baseline (speedup 1.0000x reference)
import functools

import jax
import jax.numpy as jnp
from jax import lax
from jax.experimental import pallas as pl
from jax.experimental.pallas import tpu as pltpu
from jax.experimental.pallas import tpu_sc as plsc

_SEG = 128


def _make_embed(n_rows: int, vocab: int, dim: int):
    info = plsc.get_sparse_core_info()
    nl = info.num_lanes
    nw = info.num_cores * info.num_subcores
    b_per_w = n_rows // nw
    n_seg = b_per_w // _SEG
    n_vec = dim // nl
    assert n_rows % (nw * _SEG) == 0 and n_seg % 2 == 0

    mesh = plsc.VectorSubcoreMesh(core_axis_name="c", subcore_axis_name="s")

    @functools.partial(
        pl.kernel,
        mesh=mesh,
        compiler_params=pltpu.CompilerParams(needs_layout_passes=False),
        out_type=jax.ShapeDtypeStruct((n_rows // 2, 2 * dim), jnp.float32),
        scratch_types=[
            pltpu.VMEM((b_per_w,), jnp.int32),
            pltpu.VMEM((_SEG,), jnp.int32),
            pltpu.VMEM((_SEG,), jnp.int32),
            pltpu.VMEM((2, _SEG, 2 * dim), jnp.float32),
            pltpu.VMEM((2, _SEG // 2, 2 * dim), jnp.float32),
            pltpu.SemaphoreType.DMA,
            pltpu.SemaphoreType.DMA,
            pltpu.SemaphoreType.DMA,
            pltpu.SemaphoreType.DMA,
            pltpu.SemaphoreType.DMA,
        ],
    )
    def embed(table_hbm, idx_hbm, out_hbm, idx_v, gidx0_v, gidx1_v, g_v, o_v,
              isem, gsem0, gsem1, osem0, osem1):
        gidxs = (gidx0_v, gidx1_v)
        gsems = (gsem0, gsem1)
        osems = (osem0, osem1)
        wid = lax.axis_index("s") * info.num_cores + lax.axis_index("c")
        base = pl.multiple_of(wid * b_per_w, b_per_w)

        pltpu.async_copy(idx_hbm.at[pl.ds(base, b_per_w)], idx_v, isem).wait()

        def prep_and_gather(seg, s):
            ib = pl.multiple_of(seg * _SEG, _SEG)
            for j in range(_SEG // nl):
                v = idx_v[pl.ds(ib + j * nl, nl)]
                gidxs[s][pl.ds(j * nl, nl)] = lax.shift_right_logical(v, 1)
            pltpu.async_copy(table_hbm.at[gidxs[s]], g_v.at[s], gsems[s])

        def wait_gather(s):
            pltpu.make_async_copy(
                table_hbm.at[gidxs[s]], g_v.at[s], gsems[s]).wait()

        def compact(seg, s):
            s16 = jnp.full((nl,), s, jnp.int32)
            iota = lax.iota(jnp.int32, nl)
            def body(m, _):
                for l in range(nl):
                    i = m * nl + l
                    r = m * (nl // 2) + l // 2
                    c = (l % 2) * dim
                    i16 = jnp.full((nl,), i, jnp.int32)
                    r16 = jnp.full((nl,), r, jnp.int32)
                    idx16 = plsc.load_gather(
                        idx_v, [jnp.full((nl,), seg * _SEG + i, jnp.int32)])
                    off16 = (idx16 & 1) * dim
                    for j in range(n_vec):
                        vals = plsc.load_gather(
                            g_v, [s16, i16, off16 + (iota + j * nl)])
                        plsc.store_scatter(
                            o_v, [s16, r16, iota + (c + j * nl)], vals)
                return 0
            lax.fori_loop(0, _SEG // nl, body, 0, unroll=False)

        def start_store(seg, s):
            orow = pl.multiple_of(pl.multiple_of(base // 2, b_per_w // 2) + seg * (_SEG // 2), _SEG // 2)
            pltpu.async_copy(
                o_v.at[s], out_hbm.at[pl.ds(orow, _SEG // 2)], osems[s])

        def wait_store(s):
            pltpu.make_async_copy(
                o_v.at[s], out_hbm.at[pl.ds(pl.multiple_of(base // 2, b_per_w // 2), _SEG // 2)],
                osems[s]).wait()

        prep_and_gather(0, 0)
        prep_and_gather(1, 1)

        def outer(g, _):
            for s in range(2):
                seg = g * 2 + s
                wait_gather(s)
                compact(seg, s)
                @pl.when(g > 0)
                def _():
                    wait_store(s)
                start_store(seg, s)
                @pl.when(g < n_seg // 2 - 1)
                def _():
                    prep_and_gather(seg + 2, s)
            return 0

        lax.fori_loop(0, n_seg // 2, outer, 0, unroll=False)

        wait_store(0)
        wait_store(1)

    return embed


def kernel(x, W):
    B, L = x.shape
    V, D = W.shape
    n_rows = B * L
    embed = _make_embed(n_rows, V, D)
    out = embed(W.reshape(V // 2, 2 * D), x.reshape(n_rows))
    return out.reshape(B, L, D)


# speedup vs baseline: 1.3401x; 1.3401x over previous
import functools

import jax
import jax.numpy as jnp
from jax import lax
from jax.experimental import pallas as pl
from jax.experimental.pallas import tpu as pltpu
from jax.experimental.pallas import tpu_sc as plsc

_SEG = 128
_R = 8


def _make_embed(n_rows: int, vocab: int, dim: int):
    info = plsc.get_sparse_core_info()
    nw = info.num_cores * info.num_subcores
    assert n_rows % (nw * _SEG * _R) == 0
    b_per_w = n_rows // nw
    n_seg = b_per_w // _SEG
    n_outer = n_seg // _R

    mesh = plsc.VectorSubcoreMesh(core_axis_name="c", subcore_axis_name="s")

    @functools.partial(
        pl.kernel,
        mesh=mesh,
        compiler_params=pltpu.CompilerParams(
            use_tc_tiling_on_sc=False,
            skip_device_barrier=True,
            disable_bounds_checks=True,
            disable_semaphore_checks=True,
        ),
        out_type=jax.ShapeDtypeStruct((n_rows, dim), jnp.float32),
        scratch_types=[
            pltpu.VMEM((b_per_w,), jnp.int32),
            pltpu.VMEM((_R, _SEG, dim), jnp.float32),
            pltpu.SemaphoreType.DMA,
        ] + [pltpu.SemaphoreType.DMA] * _R,
    )
    def embed(table_hbm, idx_hbm, out_hbm, idx_v, rows_v, isem, *sems):
        wid = lax.axis_index("s") * info.num_cores + lax.axis_index("c")
        base = wid * b_per_w

        pltpu.async_copy(idx_hbm.at[pl.ds(base, b_per_w)], idx_v, isem).wait()

        def gather(seg, s):
            pltpu.async_copy(
                table_hbm.at[idx_v.at[pl.ds(seg * _SEG, _SEG)]],
                rows_v.at[s], sems[s])

        def wait_slot(s):
            pltpu.make_async_copy(
                out_hbm.at[pl.ds(0, _SEG)], rows_v.at[s], sems[s]).wait()

        def store(seg, s):
            pltpu.async_copy(
                rows_v.at[s], out_hbm.at[pl.ds(base + seg * _SEG, _SEG)],
                sems[s])

        for s in range(_R):
            gather(s, s)

        def body(g, _):
            for s in range(_R):
                seg = g * _R + s
                wait_slot(s)
                store(seg, s)
                @pl.when(g < n_outer - 1)
                def _():
                    wait_slot(s)
                    gather(seg + _R, s)
            return 0

        lax.fori_loop(0, n_outer, body, 0, unroll=False)

        for s in range(_R):
            wait_slot(s)

    return embed


def kernel(x, W):
    B, L = x.shape
    V, D = W.shape
    n_rows = B * L
    embed = _make_embed(n_rows, V, D)
    out = embed(W, x.reshape(n_rows))
    return out.reshape(B, L, D)
